# baseline (speedup 1.0000x reference)
import functools

import jax
import jax.numpy as jnp
from jax import lax
from jax.experimental import pallas as pl
from jax.experimental.pallas import tpu as pltpu
from jax.experimental.pallas import tpu_sc as plsc

VOCAB = 100000
EMBED = 64
BATCH = 4096
CTX = 20

NC, NS = 2, 16
NW = NC * NS
B_PER_W = BATCH // NW
ROWS_PER_W = B_PER_W * CTX
GATHERS_PER_W = ROWS_PER_W // 128
CHUNK_B = 32
CHUNK_ROWS = CHUNK_B * CTX
GATHERS_PER_CHUNK = CHUNK_ROWS // 128
N_CHUNKS = B_PER_W // CHUNK_B
PAD_E = 128


def _sc_body(idx_hbm, table_hbm, out_hbm, idx_v, rows_v, out_v, sem0, sem1):
    wid = lax.axis_index("s") * NC + lax.axis_index("c")
    pltpu.sync_copy(idx_hbm.at[wid], idx_v)

    sems = (sem0, sem1)

    def fire(chunk, slot):
        handles = []
        for j in range(GATHERS_PER_CHUNK):
            g = chunk * GATHERS_PER_CHUNK + j
            handles.append(pltpu.async_copy(
                table_hbm.at[idx_v.at[g]],
                rows_v.at[slot].at[pl.ds(j * 128, 128)],
                sems[slot]))
        return handles

    def reduce_chunk(chunk, slot):
        base = chunk * CHUNK_B

        def body(b, carry):
            r0 = b * CTX
            for e in range(EMBED // 16):
                sl = pl.ds(e * 16, 16)
                acc = rows_v[slot, r0, sl]
                for c in range(1, CTX):
                    acc = acc + rows_v[slot, r0 + c, sl]
                out_v[pl.ds((base + b) * PAD_E + e * 16, 16)] = acc * (1.0 / CTX)
            return carry

        lax.fori_loop(0, CHUNK_B, body, 0)

    pending = fire(0, 0)
    for chunk in range(N_CHUNKS):
        slot = chunk % 2
        for h in pending:
            h.wait()
        if chunk + 1 < N_CHUNKS:
            pending = fire(chunk + 1, 1 - slot)
        reduce_chunk(chunk, slot)

    pltpu.sync_copy(out_v, out_hbm.at[pl.ds(wid * B_PER_W * PAD_E,
                                            B_PER_W * PAD_E)])


def _sc_gather_mean(idx3d, emb_table):
    mesh = plsc.VectorSubcoreMesh(core_axis_name="c", subcore_axis_name="s")
    k = pl.kernel(
        _sc_body,
        out_type=jax.ShapeDtypeStruct((BATCH * PAD_E,), jnp.float32),
        mesh=mesh,
        scratch_types=[
            pltpu.VMEM((GATHERS_PER_W, 128), jnp.int32),
            pltpu.VMEM((2, CHUNK_ROWS, EMBED), jnp.float32),
            pltpu.VMEM((B_PER_W * PAD_E,), jnp.float32),
            pltpu.SemaphoreType.DMA,
            pltpu.SemaphoreType.DMA,
        ],
        compiler_params=pltpu.CompilerParams(use_tc_tiling_on_sc=False),
    )
    return k(idx3d, emb_table)


V_BLK = 1024


def _mm_body(wt_ref, ctx_ref, b_ref, out_ref):
    ctx64 = ctx_ref[...][:, :EMBED]
    prod = lax.dot_general(
        wt_ref[...], ctx64, (((0,), (1,)), ((), ())),
        preferred_element_type=jnp.float32)
    out_ref[...] = prod + b_ref[...][:, None]


def _tc_matmul_t(WT, ctx2, b):
    nv = pl.cdiv(VOCAB, V_BLK)
    return pl.pallas_call(
        _mm_body,
        grid=(nv,),
        in_specs=[
            pl.BlockSpec((EMBED, V_BLK), lambda i: (0, i)),
            pl.BlockSpec((BATCH, PAD_E), lambda i: (0, 0)),
            pl.BlockSpec((V_BLK,), lambda i: (i,)),
        ],
        out_specs=pl.BlockSpec((V_BLK, BATCH), lambda i: (i, 0)),
        out_shape=jax.ShapeDtypeStruct((VOCAB, BATCH), jnp.float32),
        compiler_params=pltpu.CompilerParams(
            dimension_semantics=("arbitrary",),
            vmem_limit_bytes=64 * 1024 * 1024),
    )(WT, ctx2, b)


def kernel(context_words, emb_table, W, b):
    idx3d = context_words.astype(jnp.int32).reshape(NW, GATHERS_PER_W, 128)
    ctx2 = _sc_gather_mean(idx3d, emb_table).reshape(BATCH, PAD_E)
    outT = _tc_matmul_t(W.T, ctx2, b)
    return outT.T


# speedup vs baseline: 1.0010x; 1.0010x over previous
import jax
import jax.numpy as jnp
from jax import lax
from jax.experimental import pallas as pl
from jax.experimental.pallas import tpu as pltpu
from jax.experimental.pallas import tpu_sc as plsc

VOCAB = 100000
EMBED = 64
BATCH = 4096
CTX = 20

NC, NS = 2, 16
NW = NC * NS
B_PER_W = BATCH // NW
ROWS_PER_W = B_PER_W * CTX
GATHERS_PER_W = ROWS_PER_W // 128
CHUNK_B = 32
CHUNK_ROWS = CHUNK_B * CTX
GATHERS_PER_CHUNK = CHUNK_ROWS // 128
N_CHUNKS = B_PER_W // CHUNK_B
PAD_E = 128


def _sc_body(idx_hbm, table_hbm, out_hbm, idx_v, rows_v, out_v, sem0, sem1):
    wid = lax.axis_index("s") * NC + lax.axis_index("c")
    pltpu.sync_copy(idx_hbm.at[wid], idx_v)

    sems = (sem0, sem1)

    def fire(chunk, slot):
        handles = []
        for j in range(GATHERS_PER_CHUNK):
            g = chunk * GATHERS_PER_CHUNK + j
            handles.append(pltpu.async_copy(
                table_hbm.at[idx_v.at[g]],
                rows_v.at[slot].at[pl.ds(j * 128, 128)],
                sems[slot]))
        return handles

    def reduce_chunk(chunk, slot):
        base = chunk * CHUNK_B

        def body(b, carry):
            r0 = b * CTX
            for e in range(EMBED // 16):
                sl = pl.ds(e * 16, 16)
                acc = rows_v[slot, r0, sl]
                for c in range(1, CTX):
                    acc = acc + rows_v[slot, r0 + c, sl]
                out_v[pl.ds((base + b) * PAD_E + e * 16, 16)] = acc * (1.0 / CTX)
            return carry

        lax.fori_loop(0, CHUNK_B, body, 0)

    pending = fire(0, 0)
    for chunk in range(N_CHUNKS):
        slot = chunk % 2
        for h in pending:
            h.wait()
        if chunk + 1 < N_CHUNKS:
            pending = fire(chunk + 1, 1 - slot)
        reduce_chunk(chunk, slot)

    pltpu.sync_copy(out_v, out_hbm.at[pl.ds(wid * B_PER_W * PAD_E,
                                            B_PER_W * PAD_E)])


def _sc_gather_mean(idx3d, emb_table):
    mesh = plsc.VectorSubcoreMesh(core_axis_name="c", subcore_axis_name="s")
    k = pl.kernel(
        _sc_body,
        out_type=jax.ShapeDtypeStruct((BATCH * PAD_E,), jnp.float32),
        mesh=mesh,
        scratch_types=[
            pltpu.VMEM((GATHERS_PER_W, 128), jnp.int32),
            pltpu.VMEM((2, CHUNK_ROWS, EMBED), jnp.float32),
            pltpu.VMEM((B_PER_W * PAD_E,), jnp.float32),
            pltpu.SemaphoreType.DMA,
            pltpu.SemaphoreType.DMA,
        ],
        compiler_params=pltpu.CompilerParams(use_tc_tiling_on_sc=False),
    )
    return k(idx3d, emb_table)


V_BLK = 1024


def _mm_body(wt_ref, ctx_ref, b_ref, out_ref):
    ctx64 = ctx_ref[...][:, :EMBED]
    prod = lax.dot_general(
        wt_ref[...], ctx64, (((0,), (1,)), ((), ())),
        preferred_element_type=jnp.float32)
    out_ref[...] = prod + b_ref[...][:, None]


def _tc_matmul_t(WT, ctx2, b):
    nv = pl.cdiv(VOCAB, V_BLK)
    return pl.pallas_call(
        _mm_body,
        grid=(nv,),
        in_specs=[
            pl.BlockSpec((EMBED, V_BLK), lambda i: (0, i)),
            pl.BlockSpec((BATCH, PAD_E), lambda i: (0, 0)),
            pl.BlockSpec((V_BLK,), lambda i: (i,)),
        ],
        out_specs=pl.BlockSpec((V_BLK, BATCH), lambda i: (i, 0)),
        out_shape=jax.ShapeDtypeStruct((VOCAB, BATCH), jnp.float32),
        compiler_params=pltpu.CompilerParams(
            dimension_semantics=("arbitrary",),
            vmem_limit_bytes=64 * 1024 * 1024),
    )(WT, ctx2, b)


def kernel(context_words, emb_table, W, b):
    idx3d = context_words.astype(jnp.int32).reshape(NW, GATHERS_PER_W, 128)
    ctx2 = _sc_gather_mean(idx3d, emb_table).reshape(BATCH, PAD_E)
    outT = _tc_matmul_t(W.T, ctx2, b)
    return outT.T
